# baseline (speedup 1.0000x reference)
import functools

import jax
import jax.numpy as jnp
from jax import lax
from jax.experimental import pallas as pl
from jax.experimental.pallas import tpu as pltpu
from jax.experimental.pallas import tpu_sc as plsc

N_NODES = 10000
D = 128
NPAD = 10240
DEG_ROWS = NPAD // 128
E = 320000
NCORES = 2
NSUB = 16
NW = NCORES * NSUB
CHUNK = 128
CPT = (E + NW * CHUNK - 1) // (NW * CHUNK)
CPT = 80
EPAD = NW * CPT * CHUNK
ROWS_PER_TILE = NPAD // NSUB
RB = 2048

_mesh = plsc.VectorSubcoreMesh(
    core_axis_name="c", subcore_axis_name="s",
    num_cores=NCORES, num_subcores=NSUB)


SS = 8
NBLK = CPT // SS
BODY_BLKS = 2
NBODY = NBLK // BODY_BLKS


def _sc_agg_kernel(want_deg, h_hbm, src_hbm, dst_hbm, zero_hbm,
                   agg_out, *rest):
    if want_deg:
        (deg_out, srcb, dstb, rows_v, hist_v, rowidx_v, acc_sh, deg_sh,
         gsem, isem) = rest
    else:
        srcb, dstb, rows_v, acc_sh, gsem, isem = rest
    c = lax.axis_index("c")
    s = lax.axis_index("s")
    wid = s * NCORES + c
    crow0 = wid * CPT

    pltpu.sync_copy(zero_hbm.at[pl.ds(s * ROWS_PER_TILE, ROWS_PER_TILE)],
                    acc_sh.at[pl.ds(s * ROWS_PER_TILE, ROWS_PER_TILE)])
    if want_deg:
        pltpu.sync_copy(zero_hbm.at[pl.ds(0, DEG_ROWS)], hist_v)

        @pl.when(s == 0)
        def _():
            pltpu.sync_copy(zero_hbm.at[pl.ds(0, DEG_ROWS)], deg_sh)

        for i in range(DEG_ROWS // 16):
            rowidx_v[pl.ds(i * 16, 16)] = lax.iota(jnp.int32, 16) + i * 16

    plsc.subcore_barrier()

    ones16 = jnp.full((16,), 1.0, jnp.float32)

    def _fire_idx(blk, buf):
        base = crow0 + lax.min(blk, NBLK - 1) * SS
        pltpu.async_copy(src_hbm.at[pl.ds(base, SS)], srcb.at[buf], isem)
        pltpu.async_copy(dst_hbm.at[pl.ds(base, SS)], dstb.at[buf], isem)

    def _wait_idx(buf):
        pltpu.make_async_copy(src_hbm.at[pl.ds(0, SS)], srcb.at[buf],
                              isem).wait()
        pltpu.make_async_copy(dst_hbm.at[pl.ds(0, SS)], dstb.at[buf],
                              isem).wait()

    _fire_idx(0, 0)
    _wait_idx(0)
    pltpu.async_copy(h_hbm.at[srcb.at[0, 0]], rows_v.at[0], gsem)

    def body(k2, carry):
        for half in range(BODY_BLKS):
            blk = k2 * BODY_BLKS + half
            _fire_idx(blk + 1, 1 - half)
            for jj in range(SS):
                b = jj % 2
                pltpu.make_async_copy(h_hbm.at[srcb.at[half, jj]],
                                      rows_v.at[b], gsem).wait()
                if jj + 1 < SS:
                    pltpu.async_copy(h_hbm.at[srcb.at[half, jj + 1]],
                                     rows_v.at[1 - b], gsem)
                else:
                    _wait_idx(1 - half)

                    @pl.when(blk + 1 < NBLK)
                    def _():
                        pltpu.async_copy(h_hbm.at[srcb.at[1 - half, 0]],
                                         rows_v.at[1 - b], gsem)
                if want_deg:
                    for v in range(CHUNK // 16):
                        idx16 = dstb[half, jj, pl.ds(v * 16, 16)]
                        row = lax.shift_right_logical(idx16, 7)
                        col = lax.bitwise_and(idx16, 127)
                        plsc.addupdate_scatter(hist_v, [row, col], ones16)
                pltpu.sync_copy(rows_v.at[b], acc_sh.at[dstb.at[half, jj]],
                                add=True)
        return carry

    lax.fori_loop(0, NBODY, body, 0)

    if want_deg:
        pltpu.sync_copy(hist_v, deg_sh.at[rowidx_v], add=True)

    plsc.subcore_barrier()

    pltpu.sync_copy(acc_sh.at[pl.ds(s * ROWS_PER_TILE, ROWS_PER_TILE)],
                    agg_out.at[c, pl.ds(s * ROWS_PER_TILE, ROWS_PER_TILE)])

    if want_deg:
        @pl.when(s == 0)
        def _():
            pltpu.sync_copy(deg_sh, deg_out.at[c])


def _sage_agg_sc(h, srcp, dstp, zeros_hbm, want_deg):
    out_type = [jax.ShapeDtypeStruct((NCORES, NPAD, D), jnp.float32)]
    scratch_types = [
        pltpu.VMEM((2, SS, CHUNK), jnp.int32),
        pltpu.VMEM((2, SS, CHUNK), jnp.int32),
        pltpu.VMEM((2, CHUNK, D), jnp.float32),
        pltpu.VMEM_SHARED((NPAD, D), jnp.float32),
        pltpu.SemaphoreType.DMA,
        pltpu.SemaphoreType.DMA,
    ]
    if want_deg:
        out_type.append(
            jax.ShapeDtypeStruct((NCORES, DEG_ROWS, 128), jnp.float32))
        scratch_types[3:3] = [
            pltpu.VMEM((DEG_ROWS, 128), jnp.float32),
            pltpu.VMEM((DEG_ROWS,), jnp.int32),
        ]
        scratch_types.insert(6, pltpu.VMEM_SHARED((DEG_ROWS, 128),
                                                  jnp.float32))
    run = pl.kernel(functools.partial(_sc_agg_kernel, want_deg),
                    out_type=out_type, mesh=_mesh,
                    scratch_types=scratch_types,
                    compiler_params=pltpu.CompilerParams(
                        needs_layout_passes=False))
    out = run(h, srcp, dstp, zeros_hbm)
    if want_deg:
        return out[0], out[1]
    return out[0] if isinstance(out, (tuple, list)) else out


def _tc_pre(xp, W, b2d):
    def body(x_ref, w_ref, b_ref, o_ref):
        o_ref[...] = (
            jnp.dot(x_ref[...], w_ref[...], preferred_element_type=jnp.float32)
            + b_ref[...])
    return pl.pallas_call(
        body,
        grid=(NPAD // RB,),
        in_specs=[pl.BlockSpec((RB, D), lambda i: (i, 0)),
                  pl.BlockSpec((D, D), lambda i: (0, 0)),
                  pl.BlockSpec((1, D), lambda i: (0, 0))],
        out_specs=pl.BlockSpec((RB, D), lambda i: (i, 0)),
        out_shape=jax.ShapeDtypeStruct((NPAD, D), jnp.float32),
    )(xp, W, b2d)


def _mean_block(p0r, p1r, dr):
    deg = jnp.maximum(jnp.sum(dr[...], axis=0), 1.0)
    return (p0r[...] + p1r[...]) / deg


def _tc_sage_post(p0, p1, dp, h, Wl, Wr, b2d):
    def body(p0r, p1r, dr, hr, wl, wr, br, o_ref):
        mean = _mean_block(p0r, p1r, dr)
        acc = (jnp.dot(mean, wl[...], preferred_element_type=jnp.float32)
               + jnp.dot(hr[...], wr[...], preferred_element_type=jnp.float32)
               + br[...])
        o_ref[...] = jnp.maximum(acc, 0.0)
    return pl.pallas_call(
        body,
        grid=(NPAD // RB,),
        in_specs=[pl.BlockSpec((RB, D), lambda i: (i, 0)),
                  pl.BlockSpec((RB, D), lambda i: (i, 0)),
                  pl.BlockSpec((NCORES, RB, 1), lambda i: (0, i, 0)),
                  pl.BlockSpec((RB, D), lambda i: (i, 0)),
                  pl.BlockSpec((D, D), lambda i: (0, 0)),
                  pl.BlockSpec((D, D), lambda i: (0, 0)),
                  pl.BlockSpec((1, D), lambda i: (0, 0))],
        out_specs=pl.BlockSpec((RB, D), lambda i: (i, 0)),
        out_shape=jax.ShapeDtypeStruct((NPAD, D), jnp.float32),
    )(p0, p1, dp, h, Wl, Wr, b2d)


def _tc_sage_final(p0, p1, dp, h, Wl, Wr, b2d, Wp, bp2d):
    def body(p0r, p1r, dr, hr, wl, wr, br, wp, bpr, o_ref):
        mean = _mean_block(p0r, p1r, dr)
        acc = (jnp.dot(mean, wl[...], preferred_element_type=jnp.float32)
               + jnp.dot(hr[...], wr[...], preferred_element_type=jnp.float32)
               + br[...])
        h2 = jnp.maximum(acc, 0.0)
        o_ref[...] = (
            jnp.dot(h2, wp[...], preferred_element_type=jnp.float32)
            + bpr[...])
    return pl.pallas_call(
        body,
        grid=(NPAD // RB,),
        in_specs=[pl.BlockSpec((RB, D), lambda i: (i, 0)),
                  pl.BlockSpec((RB, D), lambda i: (i, 0)),
                  pl.BlockSpec((NCORES, RB, 1), lambda i: (0, i, 0)),
                  pl.BlockSpec((RB, D), lambda i: (i, 0)),
                  pl.BlockSpec((D, D), lambda i: (0, 0)),
                  pl.BlockSpec((D, D), lambda i: (0, 0)),
                  pl.BlockSpec((1, D), lambda i: (0, 0)),
                  pl.BlockSpec((D, D), lambda i: (0, 0)),
                  pl.BlockSpec((1, D), lambda i: (0, 0))],
        out_specs=pl.BlockSpec((RB, D), lambda i: (i, 0)),
        out_shape=jax.ShapeDtypeStruct((NPAD, D), jnp.float32),
    )(p0, p1, dp, h, Wl, Wr, b2d, Wp, bp2d)


def kernel(x, edge_index, W_pre, b_pre, Wl1, Wr1, b1, Wl2, Wr2, b2,
           W_post, b_post):
    src = edge_index[0].astype(jnp.int32)
    dst = edge_index[1].astype(jnp.int32)
    pad_idx = jnp.full((EPAD - E,), NPAD - 1, jnp.int32)
    srcp = jnp.concatenate([src, pad_idx]).reshape(EPAD // CHUNK, CHUNK)
    dstp = jnp.concatenate([dst, pad_idx]).reshape(EPAD // CHUNK, CHUNK)
    xp = jnp.pad(x, ((0, NPAD - N_NODES), (0, 0)))
    zeros_hbm = jnp.zeros((NPAD, D), jnp.float32)

    h0 = _tc_pre(xp, W_pre, b_pre.reshape(1, D))
    agg_p, deg_p = _sage_agg_sc(h0, srcp, dstp, zeros_hbm, True)
    dp = deg_p.reshape(NCORES, NPAD, 1)
    h1 = _tc_sage_post(agg_p[0], agg_p[1], dp, h0,
                       Wl1, Wr1, b1.reshape(1, D))
    agg2_p = _sage_agg_sc(h1, srcp, dstp, zeros_hbm, False)
    y = _tc_sage_final(agg2_p[0], agg2_p[1], dp, h1,
                       Wl2, Wr2, b2.reshape(1, D),
                       W_post, b_post.reshape(1, D))
    return y[:N_NODES]


# speedup vs baseline: 1.0680x; 1.0680x over previous
import functools

import jax
import jax.numpy as jnp
from jax import lax
from jax.experimental import pallas as pl
from jax.experimental.pallas import tpu as pltpu
from jax.experimental.pallas import tpu_sc as plsc

N_NODES = 10000
D = 128
NPAD = 10240
DEG_ROWS = NPAD // 128
E = 320000
NCORES = 2
NSUB = 16
NW = NCORES * NSUB
CHUNK = 128
CPT0 = 32
CPT1 = 128
EPAD = NSUB * (CPT0 + CPT1) * CHUNK
ROWS_PER_TILE = NPAD // NSUB
RB = 2048

_mesh = plsc.VectorSubcoreMesh(
    core_axis_name="c", subcore_axis_name="s",
    num_cores=NCORES, num_subcores=NSUB)


SS = 8
BODY_BLKS = 2


def _sc_agg_kernel(h_hbm, src_hbm, dst_hbm, zero_hbm,
                   agg_out, deg_out,
                   srcb, dstb, rows_v, hist_v, rowidx_v, acc_sh, deg_sh,
                   gsem, isem):
    c = lax.axis_index("c")
    s = lax.axis_index("s")
    is0 = c == 0
    nblk = lax.select(is0, CPT0 // SS, CPT1 // SS)
    nbody = lax.select(is0, CPT0 // (SS * BODY_BLKS),
                       CPT1 // (SS * BODY_BLKS))
    crow0 = lax.select(is0, s * CPT0, NSUB * CPT0 + s * CPT1)

    pltpu.sync_copy(zero_hbm.at[pl.ds(s * ROWS_PER_TILE, ROWS_PER_TILE)],
                    acc_sh.at[pl.ds(s * ROWS_PER_TILE, ROWS_PER_TILE)])
    pltpu.sync_copy(zero_hbm.at[pl.ds(0, DEG_ROWS)], hist_v)

    @pl.when(s == 0)
    def _():
        pltpu.sync_copy(zero_hbm.at[pl.ds(0, DEG_ROWS)], deg_sh)

    for i in range(DEG_ROWS // 16):
        rowidx_v[pl.ds(i * 16, 16)] = lax.iota(jnp.int32, 16) + i * 16

    plsc.subcore_barrier()

    ones16 = jnp.full((16,), 1.0, jnp.float32)

    def _fire_idx(blk, buf):
        base = crow0 + lax.min(blk, nblk - 1) * SS
        pltpu.async_copy(src_hbm.at[pl.ds(base, SS)], srcb.at[buf], isem)
        pltpu.async_copy(dst_hbm.at[pl.ds(base, SS)], dstb.at[buf], isem)

    def _wait_idx(buf):
        pltpu.make_async_copy(src_hbm.at[pl.ds(0, SS)], srcb.at[buf],
                              isem).wait()
        pltpu.make_async_copy(dst_hbm.at[pl.ds(0, SS)], dstb.at[buf],
                              isem).wait()

    _fire_idx(0, 0)
    _wait_idx(0)
    pltpu.async_copy(h_hbm.at[srcb.at[0, 0]], rows_v.at[0], gsem)

    def body(k2, carry):
        for half in range(BODY_BLKS):
            blk = k2 * BODY_BLKS + half
            _fire_idx(blk + 1, 1 - half)
            for jj in range(SS):
                b = jj % 2
                pltpu.make_async_copy(h_hbm.at[srcb.at[half, jj]],
                                      rows_v.at[b], gsem).wait()
                if jj + 1 < SS:
                    pltpu.async_copy(h_hbm.at[srcb.at[half, jj + 1]],
                                     rows_v.at[1 - b], gsem)
                else:
                    _wait_idx(1 - half)

                    @pl.when(blk + 1 < nblk)
                    def _():
                        pltpu.async_copy(h_hbm.at[srcb.at[1 - half, 0]],
                                         rows_v.at[1 - b], gsem)
                for v in range(CHUNK // 16):
                    idx16 = dstb[half, jj, pl.ds(v * 16, 16)]
                    row = lax.shift_right_logical(idx16, 7)
                    col = lax.bitwise_and(idx16, 127)
                    plsc.addupdate_scatter(hist_v, [row, col], ones16)
                pltpu.sync_copy(rows_v.at[b], acc_sh.at[dstb.at[half, jj]],
                                add=True)
        return carry

    lax.fori_loop(0, nbody, body, 0)

    pltpu.sync_copy(hist_v, deg_sh.at[rowidx_v], add=True)

    plsc.subcore_barrier()

    pltpu.sync_copy(acc_sh.at[pl.ds(s * ROWS_PER_TILE, ROWS_PER_TILE)],
                    agg_out.at[c, pl.ds(s * ROWS_PER_TILE, ROWS_PER_TILE)])

    @pl.when(s == 0)
    def _():
        pltpu.sync_copy(deg_sh, deg_out.at[c])


def _sage_agg_sc(h, srcp, dstp, zeros_hbm):
    out_type = [
        jax.ShapeDtypeStruct((NCORES, NPAD, D), jnp.float32),
        jax.ShapeDtypeStruct((NCORES, DEG_ROWS, 128), jnp.float32),
    ]
    scratch_types = [
        pltpu.VMEM((2, SS, CHUNK), jnp.int32),
        pltpu.VMEM((2, SS, CHUNK), jnp.int32),
        pltpu.VMEM((2, CHUNK, D), jnp.float32),
        pltpu.VMEM((DEG_ROWS, 128), jnp.float32),
        pltpu.VMEM((DEG_ROWS,), jnp.int32),
        pltpu.VMEM_SHARED((NPAD, D), jnp.float32),
        pltpu.VMEM_SHARED((DEG_ROWS, 128), jnp.float32),
        pltpu.SemaphoreType.DMA,
        pltpu.SemaphoreType.DMA,
    ]
    run = pl.kernel(_sc_agg_kernel, out_type=out_type, mesh=_mesh,
                    scratch_types=scratch_types,
                    compiler_params=pltpu.CompilerParams(
                        needs_layout_passes=False))
    return run(h, srcp, dstp, zeros_hbm)


def _tc_pre(xp, W, b2d):
    def body(x_ref, w_ref, b_ref, o_ref):
        o_ref[...] = (
            jnp.dot(x_ref[...], w_ref[...], preferred_element_type=jnp.float32)
            + b_ref[...])
    return pl.pallas_call(
        body,
        grid=(NPAD // RB,),
        in_specs=[pl.BlockSpec((RB, D), lambda i: (i, 0)),
                  pl.BlockSpec((D, D), lambda i: (0, 0)),
                  pl.BlockSpec((1, D), lambda i: (0, 0))],
        out_specs=pl.BlockSpec((RB, D), lambda i: (i, 0)),
        out_shape=jax.ShapeDtypeStruct((NPAD, D), jnp.float32),
    )(xp, W, b2d)


def _mean_block(p0r, p1r, dr):
    deg = jnp.maximum(jnp.sum(dr[...], axis=0), 1.0)
    return (p0r[...] + p1r[...]) / deg


def _tc_sage_post(p0, p1, dp, h, Wl, Wr, b2d):
    def body(p0r, p1r, dr, hr, wl, wr, br, o_ref):
        mean = _mean_block(p0r, p1r, dr)
        acc = (jnp.dot(mean, wl[...], preferred_element_type=jnp.float32)
               + jnp.dot(hr[...], wr[...], preferred_element_type=jnp.float32)
               + br[...])
        o_ref[...] = jnp.maximum(acc, 0.0)
    return pl.pallas_call(
        body,
        grid=(NPAD // RB,),
        in_specs=[pl.BlockSpec((RB, D), lambda i: (i, 0)),
                  pl.BlockSpec((RB, D), lambda i: (i, 0)),
                  pl.BlockSpec((NCORES, RB, 1), lambda i: (0, i, 0)),
                  pl.BlockSpec((RB, D), lambda i: (i, 0)),
                  pl.BlockSpec((D, D), lambda i: (0, 0)),
                  pl.BlockSpec((D, D), lambda i: (0, 0)),
                  pl.BlockSpec((1, D), lambda i: (0, 0))],
        out_specs=pl.BlockSpec((RB, D), lambda i: (i, 0)),
        out_shape=jax.ShapeDtypeStruct((NPAD, D), jnp.float32),
    )(p0, p1, dp, h, Wl, Wr, b2d)


def _tc_sage_final(p0, p1, dp, h, Wl, Wr, b2d, Wp, bp2d):
    def body(p0r, p1r, dr, hr, wl, wr, br, wp, bpr, o_ref):
        mean = _mean_block(p0r, p1r, dr)
        acc = (jnp.dot(mean, wl[...], preferred_element_type=jnp.float32)
               + jnp.dot(hr[...], wr[...], preferred_element_type=jnp.float32)
               + br[...])
        h2 = jnp.maximum(acc, 0.0)
        o_ref[...] = (
            jnp.dot(h2, wp[...], preferred_element_type=jnp.float32)
            + bpr[...])
    return pl.pallas_call(
        body,
        grid=(NPAD // RB,),
        in_specs=[pl.BlockSpec((RB, D), lambda i: (i, 0)),
                  pl.BlockSpec((RB, D), lambda i: (i, 0)),
                  pl.BlockSpec((NCORES, RB, 1), lambda i: (0, i, 0)),
                  pl.BlockSpec((RB, D), lambda i: (i, 0)),
                  pl.BlockSpec((D, D), lambda i: (0, 0)),
                  pl.BlockSpec((D, D), lambda i: (0, 0)),
                  pl.BlockSpec((1, D), lambda i: (0, 0)),
                  pl.BlockSpec((D, D), lambda i: (0, 0)),
                  pl.BlockSpec((1, D), lambda i: (0, 0))],
        out_specs=pl.BlockSpec((RB, D), lambda i: (i, 0)),
        out_shape=jax.ShapeDtypeStruct((NPAD, D), jnp.float32),
    )(p0, p1, dp, h, Wl, Wr, b2d, Wp, bp2d)


def kernel(x, edge_index, W_pre, b_pre, Wl1, Wr1, b1, Wl2, Wr2, b2,
           W_post, b_post):
    src = edge_index[0].astype(jnp.int32)
    dst = edge_index[1].astype(jnp.int32)
    pad_idx = jnp.full((EPAD - E,), NPAD - 1, jnp.int32)
    srcp = jnp.concatenate([src, pad_idx]).reshape(EPAD // CHUNK, CHUNK)
    dstp = jnp.concatenate([dst, pad_idx]).reshape(EPAD // CHUNK, CHUNK)
    xp = jnp.pad(x, ((0, NPAD - N_NODES), (0, 0)))
    zeros_hbm = jnp.zeros((NPAD, D), jnp.float32)

    h0 = _tc_pre(xp, W_pre, b_pre.reshape(1, D))
    agg_p, deg_p = _sage_agg_sc(h0, srcp, dstp, zeros_hbm)
    dp = deg_p.reshape(NCORES, NPAD, 1)
    h1 = _tc_sage_post(agg_p[0], agg_p[1], dp, h0,
                       Wl1, Wr1, b1.reshape(1, D))
    agg2_p, _ = _sage_agg_sc(h1, srcp, dstp, zeros_hbm)
    y = _tc_sage_final(agg2_p[0], agg2_p[1], dp, h1,
                       Wl2, Wr2, b2.reshape(1, D),
                       W_post, b_post.reshape(1, D))
    return y[:N_NODES]


# speedup vs baseline: 1.1590x; 1.0852x over previous
import functools

import jax
import jax.numpy as jnp
from jax import lax
from jax.experimental import pallas as pl
from jax.experimental.pallas import tpu as pltpu
from jax.experimental.pallas import tpu_sc as plsc

N_NODES = 10000
D = 128
NPAD = 10240
DEG_ROWS = NPAD // 128
E = 320000
NCORES = 2
NSUB = 16
NW = NCORES * NSUB
CHUNK = 128
CPT = (E + NW * CHUNK - 1) // (NW * CHUNK)
CPT = 80
EPAD = NW * CPT * CHUNK
ROWS_PER_TILE = NPAD // NSUB
RB = 2048

_mesh = plsc.VectorSubcoreMesh(
    core_axis_name="c", subcore_axis_name="s",
    num_cores=NCORES, num_subcores=NSUB)


SS = 8
NBLK = CPT // SS
BODY_BLKS = 2
NBODY = NBLK // BODY_BLKS


def _sc_agg_kernel(h_hbm, src_hbm, dst_hbm, zero_hbm,
                   agg_out, deg_out,
                   srcb, dstb, rows_v, hist_v, rowidx_v, acc_sh, deg_sh,
                   gsem, isem):
    c = lax.axis_index("c")
    s = lax.axis_index("s")
    wid = s * NCORES + c
    crow0 = wid * CPT

    pltpu.sync_copy(zero_hbm.at[pl.ds(s * ROWS_PER_TILE, ROWS_PER_TILE)],
                    acc_sh.at[pl.ds(s * ROWS_PER_TILE, ROWS_PER_TILE)])
    pltpu.sync_copy(zero_hbm.at[pl.ds(0, DEG_ROWS)], hist_v)

    @pl.when(s == 0)
    def _():
        pltpu.sync_copy(zero_hbm.at[pl.ds(0, DEG_ROWS)], deg_sh)

    for i in range(DEG_ROWS // 16):
        rowidx_v[pl.ds(i * 16, 16)] = lax.iota(jnp.int32, 16) + i * 16

    plsc.subcore_barrier()

    ones16 = jnp.full((16,), 1.0, jnp.float32)

    def _fire_idx(blk, buf):
        base = crow0 + lax.min(blk, NBLK - 1) * SS
        pltpu.async_copy(src_hbm.at[pl.ds(base, SS)], srcb.at[buf], isem)
        pltpu.async_copy(dst_hbm.at[pl.ds(base, SS)], dstb.at[buf], isem)

    def _wait_idx(buf):
        pltpu.make_async_copy(src_hbm.at[pl.ds(0, SS)], srcb.at[buf],
                              isem).wait()
        pltpu.make_async_copy(dst_hbm.at[pl.ds(0, SS)], dstb.at[buf],
                              isem).wait()

    _fire_idx(0, 0)
    _wait_idx(0)
    pltpu.async_copy(h_hbm.at[srcb.at[0, 0]], rows_v.at[0], gsem)

    def body(k2, carry):
        for half in range(BODY_BLKS):
            blk = k2 * BODY_BLKS + half
            _fire_idx(blk + 1, 1 - half)
            for jj in range(SS):
                b = jj % 2
                pltpu.make_async_copy(h_hbm.at[srcb.at[half, jj]],
                                      rows_v.at[b], gsem).wait()
                if jj + 1 < SS:
                    pltpu.async_copy(h_hbm.at[srcb.at[half, jj + 1]],
                                     rows_v.at[1 - b], gsem)
                else:
                    _wait_idx(1 - half)

                    @pl.when(blk + 1 < NBLK)
                    def _():
                        pltpu.async_copy(h_hbm.at[srcb.at[1 - half, 0]],
                                         rows_v.at[1 - b], gsem)
                for v in range(CHUNK // 16):
                    idx16 = dstb[half, jj, pl.ds(v * 16, 16)]
                    row = lax.shift_right_logical(idx16, 7)
                    col = lax.bitwise_and(idx16, 127)
                    plsc.addupdate_scatter(hist_v, [row, col], ones16)
                pltpu.sync_copy(rows_v.at[b], acc_sh.at[dstb.at[half, jj]],
                                add=True)
        return carry

    lax.fori_loop(0, NBODY, body, 0)

    pltpu.sync_copy(hist_v, deg_sh.at[rowidx_v], add=True)

    plsc.subcore_barrier()

    pltpu.sync_copy(acc_sh.at[pl.ds(s * ROWS_PER_TILE, ROWS_PER_TILE)],
                    agg_out.at[c, pl.ds(s * ROWS_PER_TILE, ROWS_PER_TILE)])

    @pl.when(s == 0)
    def _():
        pltpu.sync_copy(deg_sh, deg_out.at[c])


def _sage_agg_sc(h, srcp, dstp, zeros_hbm):
    out_type = [
        jax.ShapeDtypeStruct((NCORES, NPAD, D), jnp.float32),
        jax.ShapeDtypeStruct((NCORES, DEG_ROWS, 128), jnp.float32),
    ]
    scratch_types = [
        pltpu.VMEM((2, SS, CHUNK), jnp.int32),
        pltpu.VMEM((2, SS, CHUNK), jnp.int32),
        pltpu.VMEM((2, CHUNK, D), jnp.float32),
        pltpu.VMEM((DEG_ROWS, 128), jnp.float32),
        pltpu.VMEM((DEG_ROWS,), jnp.int32),
        pltpu.VMEM_SHARED((NPAD, D), jnp.float32),
        pltpu.VMEM_SHARED((DEG_ROWS, 128), jnp.float32),
        pltpu.SemaphoreType.DMA,
        pltpu.SemaphoreType.DMA,
    ]
    run = pl.kernel(_sc_agg_kernel, out_type=out_type, mesh=_mesh,
                    scratch_types=scratch_types,
                    compiler_params=pltpu.CompilerParams(
                        needs_layout_passes=False))
    return run(h, srcp, dstp, zeros_hbm)


def _tc_pre(xp, W, b2d):
    def body(x_ref, w_ref, b_ref, o_ref):
        o_ref[...] = (
            jnp.dot(x_ref[...], w_ref[...], preferred_element_type=jnp.float32)
            + b_ref[...])
    return pl.pallas_call(
        body,
        grid=(NPAD // RB,),
        in_specs=[pl.BlockSpec((RB, D), lambda i: (i, 0)),
                  pl.BlockSpec((D, D), lambda i: (0, 0)),
                  pl.BlockSpec((1, D), lambda i: (0, 0))],
        out_specs=pl.BlockSpec((RB, D), lambda i: (i, 0)),
        out_shape=jax.ShapeDtypeStruct((NPAD, D), jnp.float32),
    )(xp, W, b2d)


def _mean_block(p0r, p1r, dr):
    deg = jnp.maximum(jnp.sum(dr[...], axis=0), 1.0)
    return (p0r[...] + p1r[...]) / deg


def _tc_sage_post(p0, p1, dp, h, Wl, Wr, b2d):
    def body(p0r, p1r, dr, hr, wl, wr, br, o_ref):
        mean = _mean_block(p0r, p1r, dr)
        acc = (jnp.dot(mean, wl[...], preferred_element_type=jnp.float32)
               + jnp.dot(hr[...], wr[...], preferred_element_type=jnp.float32)
               + br[...])
        o_ref[...] = jnp.maximum(acc, 0.0)
    return pl.pallas_call(
        body,
        grid=(NPAD // RB,),
        in_specs=[pl.BlockSpec((RB, D), lambda i: (i, 0)),
                  pl.BlockSpec((RB, D), lambda i: (i, 0)),
                  pl.BlockSpec((NCORES, RB, 1), lambda i: (0, i, 0)),
                  pl.BlockSpec((RB, D), lambda i: (i, 0)),
                  pl.BlockSpec((D, D), lambda i: (0, 0)),
                  pl.BlockSpec((D, D), lambda i: (0, 0)),
                  pl.BlockSpec((1, D), lambda i: (0, 0))],
        out_specs=pl.BlockSpec((RB, D), lambda i: (i, 0)),
        out_shape=jax.ShapeDtypeStruct((NPAD, D), jnp.float32),
    )(p0, p1, dp, h, Wl, Wr, b2d)


def _tc_sage_final(p0, p1, dp, h, Wl, Wr, b2d, Wp, bp2d):
    def body(p0r, p1r, dr, hr, wl, wr, br, wp, bpr, o_ref):
        mean = _mean_block(p0r, p1r, dr)
        acc = (jnp.dot(mean, wl[...], preferred_element_type=jnp.float32)
               + jnp.dot(hr[...], wr[...], preferred_element_type=jnp.float32)
               + br[...])
        h2 = jnp.maximum(acc, 0.0)
        o_ref[...] = (
            jnp.dot(h2, wp[...], preferred_element_type=jnp.float32)
            + bpr[...])
    return pl.pallas_call(
        body,
        grid=(NPAD // RB,),
        in_specs=[pl.BlockSpec((RB, D), lambda i: (i, 0)),
                  pl.BlockSpec((RB, D), lambda i: (i, 0)),
                  pl.BlockSpec((NCORES, RB, 1), lambda i: (0, i, 0)),
                  pl.BlockSpec((RB, D), lambda i: (i, 0)),
                  pl.BlockSpec((D, D), lambda i: (0, 0)),
                  pl.BlockSpec((D, D), lambda i: (0, 0)),
                  pl.BlockSpec((1, D), lambda i: (0, 0)),
                  pl.BlockSpec((D, D), lambda i: (0, 0)),
                  pl.BlockSpec((1, D), lambda i: (0, 0))],
        out_specs=pl.BlockSpec((RB, D), lambda i: (i, 0)),
        out_shape=jax.ShapeDtypeStruct((NPAD, D), jnp.float32),
    )(p0, p1, dp, h, Wl, Wr, b2d, Wp, bp2d)


def kernel(x, edge_index, W_pre, b_pre, Wl1, Wr1, b1, Wl2, Wr2, b2,
           W_post, b_post):
    src = edge_index[0].astype(jnp.int32)
    dst = edge_index[1].astype(jnp.int32)
    pad_idx = jnp.full((EPAD - E,), NPAD - 1, jnp.int32)
    srcp = jnp.concatenate([src, pad_idx]).reshape(EPAD // CHUNK, CHUNK)
    dstp = jnp.concatenate([dst, pad_idx]).reshape(EPAD // CHUNK, CHUNK)
    xp = jnp.pad(x, ((0, NPAD - N_NODES), (0, 0)))
    zeros_hbm = jnp.zeros((NPAD, D), jnp.float32)

    h0 = _tc_pre(xp, W_pre, b_pre.reshape(1, D))
    agg_p, deg_p = _sage_agg_sc(h0, srcp, dstp, zeros_hbm)
    dp = deg_p.reshape(NCORES, NPAD, 1)
    h1 = _tc_sage_post(agg_p[0], agg_p[1], dp, h0,
                       Wl1, Wr1, b1.reshape(1, D))
    agg2_p, _ = _sage_agg_sc(h1, srcp, dstp, zeros_hbm)
    y = _tc_sage_final(agg2_p[0], agg2_p[1], dp, h1,
                       Wl2, Wr2, b2.reshape(1, D),
                       W_post, b_post.reshape(1, D))
    return y[:N_NODES]
